# Initial kernel scaffold; baseline (speedup 1.0000x reference)
import jax
import jax.numpy as jnp
from jax.experimental import pallas as pl


def kernel(x, edge_index, W1, b1, W2, b2):
    raise NotImplementedError("write your pallas kernel here")


# speedup vs baseline: 16.1464x; 16.1464x over previous
import jax
import jax.numpy as jnp
from jax import lax
from jax.experimental import pallas as pl
from jax.experimental.pallas import tpu as pltpu
from jax.experimental.pallas import tpu_sc as plsc

N = 10000
E = 320000
D = 128

NC = 2
NS = 16
NW = NC * NS

CHUNK = 128
NCH = E // CHUNK
RPW = 624
TAIL = N - RPW * NS
ZROWS = 208
DEGW = 16

_sc_mesh = plsc.VectorSubcoreMesh(
    core_axis_name="c", subcore_axis_name="s", num_cores=NC, num_subcores=NS
)


def _deg_body(dst_hbm, out_hbm, idx_v, ones_v, zero_v, acc):
    c = lax.axis_index("c")
    s = lax.axis_index("s")
    wid = s * NC + c

    @pl.loop(0, CHUNK)
    def _fill_ones(i):
        ones_v[i, :] = jnp.ones((16,), jnp.float32)

    @pl.loop(0, ZROWS)
    def _fill_zero(i):
        zero_v[i, :] = jnp.zeros((16,), jnp.float32)

    row0 = pl.multiple_of(s * RPW, 8)
    for k in range(RPW // ZROWS):
        pltpu.sync_copy(zero_v, acc.at[pl.ds(row0 + k * ZROWS, ZROWS)])

    @pl.when(s == NS - 1)
    def _tail_zero():
        pltpu.sync_copy(zero_v.at[pl.ds(0, TAIL)], acc.at[pl.ds(RPW * NS, TAIL)])

    plsc.subcore_barrier()

    @pl.loop(wid, NCH, step=NW)
    def _chunk(g):
        base = pl.multiple_of(g * CHUNK, CHUNK)
        pltpu.sync_copy(dst_hbm.at[pl.ds(base, CHUNK)], idx_v)
        pltpu.sync_copy(ones_v, acc.at[idx_v], add=True)

    plsc.subcore_barrier()
    pltpu.sync_copy(acc.at[pl.ds(row0, RPW)], out_hbm.at[c, pl.ds(row0, RPW)])

    @pl.when(s == NS - 1)
    def _tail_out():
        pltpu.sync_copy(acc.at[pl.ds(RPW * NS, TAIL)],
                        out_hbm.at[c, pl.ds(RPW * NS, TAIL)])


_deg_call = pl.kernel(
    _deg_body,
    out_type=jax.ShapeDtypeStruct((NC, N, DEGW), jnp.float32),
    mesh=_sc_mesh,
    scratch_types=[
        pltpu.VMEM((CHUNK,), jnp.int32),
        pltpu.VMEM((CHUNK, DEGW), jnp.float32),
        pltpu.VMEM((ZROWS, DEGW), jnp.float32),
        pltpu.VMEM_SHARED((N, DEGW), jnp.float32),
    ],
)


def _agg_body(y_hbm, src_hbm, dst_hbm, out_hbm, idx_s, idx_d, rows, zrows, acc):
    c = lax.axis_index("c")
    s = lax.axis_index("s")
    wid = s * NC + c

    @pl.loop(0, ZROWS)
    def _fill_zero(i):
        for j in range(D // 16):
            zrows[i, pl.ds(j * 16, 16)] = jnp.zeros((16,), jnp.float32)

    row0 = pl.multiple_of(s * RPW, 8)
    for k in range(RPW // ZROWS):
        pltpu.sync_copy(zrows, acc.at[pl.ds(row0 + k * ZROWS, ZROWS)])

    @pl.when(s == NS - 1)
    def _tail_zero():
        pltpu.sync_copy(zrows.at[pl.ds(0, TAIL)], acc.at[pl.ds(RPW * NS, TAIL)])

    plsc.subcore_barrier()

    @pl.loop(wid, NCH, step=NW)
    def _chunk(g):
        base = pl.multiple_of(g * CHUNK, CHUNK)
        pltpu.sync_copy(src_hbm.at[pl.ds(base, CHUNK)], idx_s)
        pltpu.sync_copy(dst_hbm.at[pl.ds(base, CHUNK)], idx_d)
        pltpu.sync_copy(y_hbm.at[idx_s], rows)
        pltpu.sync_copy(rows, acc.at[idx_d], add=True)

    plsc.subcore_barrier()
    pltpu.sync_copy(acc.at[pl.ds(row0, RPW)], out_hbm.at[c, pl.ds(row0, RPW)])

    @pl.when(s == NS - 1)
    def _tail_out():
        pltpu.sync_copy(acc.at[pl.ds(RPW * NS, TAIL)],
                        out_hbm.at[c, pl.ds(RPW * NS, TAIL)])


_agg_call = pl.kernel(
    _agg_body,
    out_type=jax.ShapeDtypeStruct((NC, N, D), jnp.float32),
    mesh=_sc_mesh,
    scratch_types=[
        pltpu.VMEM((CHUNK,), jnp.int32),
        pltpu.VMEM((CHUNK,), jnp.int32),
        pltpu.VMEM((CHUNK, D), jnp.float32),
        pltpu.VMEM((ZROWS, D), jnp.float32),
        pltpu.VMEM_SHARED((N, D), jnp.float32),
    ],
)


BLK = 1000
NBLK = N // BLK


def _dis_from(dp):
    return lax.rsqrt(1.0 + dp[0, :, 0:1] + dp[1, :, 0:1])


def _mm_scale_body(dp_ref, x_ref, w_ref, y_ref):
    dis = _dis_from(dp_ref[...])
    y_ref[...] = jnp.dot(x_ref[...], w_ref[...],
                         preferred_element_type=jnp.float32) * dis


def _layer2_body(dp_ref, y1_ref, p_ref, b1_ref, w2_ref, y2_ref):
    dis = _dis_from(dp_ref[...])
    pe = p_ref[...]
    h = (y1_ref[...] + pe[0] + pe[1]) * dis + b1_ref[...]
    h = jnp.maximum(h, 0.0)
    y2_ref[...] = jnp.dot(h, w2_ref[...],
                          preferred_element_type=jnp.float32) * dis


def _final_body(dp_ref, y2_ref, q_ref, b2_ref, o_ref):
    dis = _dis_from(dp_ref[...])
    qe = q_ref[...]
    o_ref[...] = (y2_ref[...] + qe[0] + qe[1]) * dis + b2_ref[...]


_dp_spec = pl.BlockSpec((NC, BLK, DEGW), lambda i: (0, i, 0))
_row_spec = pl.BlockSpec((BLK, D), lambda i: (i, 0))
_par_spec = pl.BlockSpec((NC, BLK, D), lambda i: (0, i, 0))
_w_spec = pl.BlockSpec((D, D), lambda i: (0, 0))
_b_spec = pl.BlockSpec((1, D), lambda i: (0, 0))
_out_struct = jax.ShapeDtypeStruct((N, D), jnp.float32)

_mm_scale = pl.pallas_call(
    _mm_scale_body,
    grid=(NBLK,),
    in_specs=[_dp_spec, _row_spec, _w_spec],
    out_specs=_row_spec,
    out_shape=_out_struct,
)

_layer2 = pl.pallas_call(
    _layer2_body,
    grid=(NBLK,),
    in_specs=[_dp_spec, _row_spec, _par_spec, _b_spec, _w_spec],
    out_specs=_row_spec,
    out_shape=_out_struct,
)

_final = pl.pallas_call(
    _final_body,
    grid=(NBLK,),
    in_specs=[_dp_spec, _row_spec, _par_spec, _b_spec],
    out_specs=_row_spec,
    out_shape=_out_struct,
)


def kernel(x, edge_index, W1, b1, W2, b2):
    src = edge_index[0]
    dst = edge_index[1]
    dp = _deg_call(dst)
    y1 = _mm_scale(dp, x, W1)
    p = _agg_call(y1, src, dst)
    y2 = _layer2(dp, y1, p, b1.reshape(1, D), W2)
    q = _agg_call(y2, src, dst)
    return _final(dp, y2, q, b2.reshape(1, D))


# speedup vs baseline: 23.8181x; 1.4751x over previous
import jax
import jax.numpy as jnp
from jax import lax
from jax.experimental import pallas as pl
from jax.experimental.pallas import tpu as pltpu
from jax.experimental.pallas import tpu_sc as plsc

N = 10000
E = 320000
D = 128

NC = 2
NS = 16
NW = NC * NS

CHUNK = 128
NCH = E // CHUNK
NFULL = NCH // NW
NEXTRA = NCH % NW
RPW = 624
TAIL = N - RPW * NS
ZROWS = 208
DEGW = 16

_sc_mesh = plsc.VectorSubcoreMesh(
    core_axis_name="c", subcore_axis_name="s", num_cores=NC, num_subcores=NS
)


def _deg_body(dst_hbm, out_hbm, idx_v, ones_v, zero_v, acc):
    c = lax.axis_index("c")
    s = lax.axis_index("s")
    wid = s * NC + c

    @pl.loop(0, CHUNK)
    def _fill_ones(i):
        ones_v[i, :] = jnp.ones((16,), jnp.float32)

    @pl.loop(0, ZROWS)
    def _fill_zero(i):
        zero_v[i, :] = jnp.zeros((16,), jnp.float32)

    row0 = pl.multiple_of(s * RPW, 8)
    for k in range(RPW // ZROWS):
        pltpu.sync_copy(zero_v, acc.at[pl.ds(row0 + k * ZROWS, ZROWS)])

    @pl.when(s == NS - 1)
    def _tail_zero():
        pltpu.sync_copy(zero_v.at[pl.ds(0, TAIL)], acc.at[pl.ds(RPW * NS, TAIL)])

    plsc.subcore_barrier()

    @pl.loop(wid, NCH, step=NW)
    def _chunk(g):
        base = pl.multiple_of(g * CHUNK, CHUNK)
        pltpu.sync_copy(dst_hbm.at[pl.ds(base, CHUNK)], idx_v)
        pltpu.sync_copy(ones_v, acc.at[idx_v], add=True)

    plsc.subcore_barrier()
    pltpu.sync_copy(acc.at[pl.ds(row0, RPW)], out_hbm.at[c, pl.ds(row0, RPW)])

    @pl.when(s == NS - 1)
    def _tail_out():
        pltpu.sync_copy(acc.at[pl.ds(RPW * NS, TAIL)],
                        out_hbm.at[c, pl.ds(RPW * NS, TAIL)])


_deg_call = pl.kernel(
    _deg_body,
    out_type=jax.ShapeDtypeStruct((NC, N, DEGW), jnp.float32),
    mesh=_sc_mesh,
    scratch_types=[
        pltpu.VMEM((CHUNK,), jnp.int32),
        pltpu.VMEM((CHUNK, DEGW), jnp.float32),
        pltpu.VMEM((ZROWS, DEGW), jnp.float32),
        pltpu.VMEM_SHARED((N, DEGW), jnp.float32),
    ],
)


def _agg_body(y_hbm, src_hbm, dst_hbm, out_hbm, is0, is1, idx_d, rows,
              acc, sem0, sem1):
    c = lax.axis_index("c")
    s = lax.axis_index("s")
    wid = s * NC + c

    @pl.loop(0, CHUNK)
    def _fill_zero(i):
        for j in range(D // 16):
            rows[0, i, pl.ds(j * 16, 16)] = jnp.zeros((16,), jnp.float32)

    row0 = pl.multiple_of(s * RPW, 8)
    for k in range(RPW // CHUNK):
        pltpu.sync_copy(rows.at[0], acc.at[pl.ds(row0 + k * CHUNK, CHUNK)])
    rem = RPW % CHUNK
    pltpu.sync_copy(rows.at[0, pl.ds(0, rem)],
                    acc.at[pl.ds(row0 + RPW - rem, rem)])

    @pl.when(s == NS - 1)
    def _tail_zero():
        pltpu.sync_copy(rows.at[0, pl.ds(0, TAIL)],
                        acc.at[pl.ds(RPW * NS, TAIL)])

    plsc.subcore_barrier()

    isb = (is0, is1)
    sems = (sem0, sem1)

    def _ebase(t):
        return pl.multiple_of((wid + NW * t) * CHUNK, CHUNK)

    def _issue(t, b):
        pltpu.sync_copy(src_hbm.at[pl.ds(_ebase(t), CHUNK)], isb[b])
        pltpu.async_copy(y_hbm.at[isb[b]], rows.at[b], sems[b])

    def _finish(t, b):
        pltpu.make_async_copy(y_hbm.at[isb[b]], rows.at[b], sems[b]).wait()
        pltpu.sync_copy(dst_hbm.at[pl.ds(_ebase(t), CHUNK)], idx_d)
        pltpu.sync_copy(rows.at[b], acc.at[idx_d], add=True)

    _issue(0, 0)
    _issue(1, 1)

    @pl.loop(0, NFULL - 2, step=2)
    def _chunk(t):
        for b in range(2):
            _finish(t + b, b)
            _issue(t + b + 2, b)

    _finish(NFULL - 2, 0)

    @pl.when(wid < NEXTRA)
    def _issue_extra():
        _issue(NFULL, 0)

    _finish(NFULL - 1, 1)

    @pl.when(wid < NEXTRA)
    def _finish_extra():
        _finish(NFULL, 0)

    plsc.subcore_barrier()
    pltpu.sync_copy(acc.at[pl.ds(row0, RPW)], out_hbm.at[c, pl.ds(row0, RPW)])

    @pl.when(s == NS - 1)
    def _tail_out():
        pltpu.sync_copy(acc.at[pl.ds(RPW * NS, TAIL)],
                        out_hbm.at[c, pl.ds(RPW * NS, TAIL)])


_agg_call = pl.kernel(
    _agg_body,
    out_type=jax.ShapeDtypeStruct((NC, N, D), jnp.float32),
    mesh=_sc_mesh,
    scratch_types=[
        pltpu.VMEM((CHUNK,), jnp.int32),
        pltpu.VMEM((CHUNK,), jnp.int32),
        pltpu.VMEM((CHUNK,), jnp.int32),
        pltpu.VMEM((2, CHUNK, D), jnp.float32),
        pltpu.VMEM_SHARED((N, D), jnp.float32),
        pltpu.SemaphoreType.DMA,
        pltpu.SemaphoreType.DMA,
    ],
)


BLK = 1000
NBLK = N // BLK


def _dis_from(dp):
    return lax.rsqrt(1.0 + dp[0, :, 0:1] + dp[1, :, 0:1])


def _mm_scale_body(dp_ref, x_ref, w_ref, y_ref):
    dis = _dis_from(dp_ref[...])
    y_ref[...] = jnp.dot(x_ref[...], w_ref[...],
                         preferred_element_type=jnp.float32) * dis


def _layer2_body(dp_ref, y1_ref, p_ref, b1_ref, w2_ref, y2_ref):
    dis = _dis_from(dp_ref[...])
    pe = p_ref[...]
    h = (y1_ref[...] + pe[0] + pe[1]) * dis + b1_ref[...]
    h = jnp.maximum(h, 0.0)
    y2_ref[...] = jnp.dot(h, w2_ref[...],
                          preferred_element_type=jnp.float32) * dis


def _final_body(dp_ref, y2_ref, q_ref, b2_ref, o_ref):
    dis = _dis_from(dp_ref[...])
    qe = q_ref[...]
    o_ref[...] = (y2_ref[...] + qe[0] + qe[1]) * dis + b2_ref[...]


_dp_spec = pl.BlockSpec((NC, BLK, DEGW), lambda i: (0, i, 0))
_row_spec = pl.BlockSpec((BLK, D), lambda i: (i, 0))
_par_spec = pl.BlockSpec((NC, BLK, D), lambda i: (0, i, 0))
_w_spec = pl.BlockSpec((D, D), lambda i: (0, 0))
_b_spec = pl.BlockSpec((1, D), lambda i: (0, 0))
_out_struct = jax.ShapeDtypeStruct((N, D), jnp.float32)

_mm_scale = pl.pallas_call(
    _mm_scale_body,
    grid=(NBLK,),
    in_specs=[_dp_spec, _row_spec, _w_spec],
    out_specs=_row_spec,
    out_shape=_out_struct,
)

_layer2 = pl.pallas_call(
    _layer2_body,
    grid=(NBLK,),
    in_specs=[_dp_spec, _row_spec, _par_spec, _b_spec, _w_spec],
    out_specs=_row_spec,
    out_shape=_out_struct,
)

_final = pl.pallas_call(
    _final_body,
    grid=(NBLK,),
    in_specs=[_dp_spec, _row_spec, _par_spec, _b_spec],
    out_specs=_row_spec,
    out_shape=_out_struct,
)


def kernel(x, edge_index, W1, b1, W2, b2):
    src = edge_index[0]
    dst = edge_index[1]
    dp = _deg_call(dst)
    y1 = _mm_scale(dp, x, W1)
    p = _agg_call(y1, src, dst)
    y2 = _layer2(dp, y1, p, b1.reshape(1, D), W2)
    q = _agg_call(y2, src, dst)
    return _final(dp, y2, q, b2.reshape(1, D))
